# baseline (speedup 1.0000x reference)
import jax
import jax.numpy as jnp
from jax import lax
from jax.experimental import pallas as pl
from jax.experimental.pallas import tpu as pltpu
from jax.experimental.pallas import tpu_sc as plsc

NE = 8
H = 2048
F = 768
K = 2
N = 4096
NPAIR = N * K
BT = 256
ROWS = NPAIR + NE * BT
G = ROWS // BT
GBUF = 64

NC = 2
NS = 16
L = 16
RPT = ROWS // NS

_i32 = jnp.int32
_f32 = jnp.float32
_SC_PARAMS = pltpu.CompilerParams(needs_layout_passes=False)


def _route_body(ri_hbm, rw_hbm, w01_hbm, pos_hbm, be_hbm,
                ri_full, ri_v, rw_v, pos_v, wv_v,
                cnt16_v, mybase_v, pb_v, cumblk_v, bev):
    cid = lax.axis_index("c")
    sid = lax.axis_index("s")
    iota = lax.iota(_i32, L)
    ones_i = jnp.ones((L,), _i32)
    zeros_i = jnp.zeros((L,), _i32)
    zeros_f = jnp.zeros((L,), _f32)

    @pl.when(cid == 0)
    def _():
        tpt = N // NS
        pltpu.sync_copy(ri_hbm, ri_full)
        pltpu.sync_copy(ri_hbm.at[:, pl.ds(sid * tpt, tpt)], ri_v)
        pltpu.sync_copy(rw_hbm.at[pl.ds(sid * tpt, tpt), :], rw_v)
        for s2 in range(NS):
            cnt16_v[s2] = zeros_i
        for s2 in range(NS):
            for k in range(K):
                for c in range(tpt // L):
                    ev = ri_full[k, pl.ds(s2 * tpt + c * L, L)]
                    plsc.addupdate_scatter(cnt16_v.at[s2], [ev], ones_i)

        total = zeros_i
        before = zeros_i
        for s2 in range(NS):
            row = cnt16_v[s2]
            total = total + row
            before = before + jnp.where(s2 < sid, row, zeros_i)
        padded = jnp.bitwise_and(total + (BT - 1), jnp.full((L,), -BT, _i32))
        cum = plsc.cumsum(padded)
        start = cum - padded
        mybase_v[...] = start + before
        pb_v[...] = lax.shift_right_logical(start, 8)
        cumblk_v[...] = lax.shift_right_logical(cum, 8)

        @pl.when(sid == 0)
        def _():
            nblk = plsc.load_gather(cumblk_v, [jnp.full((L,), 7, _i32)])
            for cb in range(GBUF // L):
                g_vec = iota + cb * L
                acc = zeros_i
                for e in range(1, NE):
                    pbe = plsc.load_gather(pb_v, [jnp.full((L,), e, _i32)])
                    acc = acc + (g_vec >= pbe).astype(_i32)
                acc = jnp.where(g_vec == 48, nblk, acc)
                bev[pl.ds(cb * L, L)] = acc
            pltpu.sync_copy(bev, be_hbm)

        for k in range(K):
            for c in range(tpt // L):
                ev = ri_v[k, pl.ds(c * L, L)]
                base = plsc.load_gather(mybase_v, [ev])
                r = zeros_i
                for e in range(NE):
                    mi = (ev == e).astype(_i32)
                    r = r + mi * (plsc.cumsum(mi) - 1)
                dest = jnp.clip(base + r, 0, ROWS - 1)
                plsc.addupdate_scatter(mybase_v, [ev], ones_i)
                pos_v[k, pl.ds(c * L, L)] = dest
                tokl = iota + c * L
                w = plsc.load_gather(rw_v, [tokl, ev])
                if k == 1:
                    ev0 = ri_v[0, pl.ds(c * L, L)]
                    w = jnp.where(ev != ev0, w, 0.0)
                wv_v[k, pl.ds(c * L, L)] = w
        pltpu.sync_copy(pos_v, pos_hbm.at[:, pl.ds(sid * tpt, tpt)])
        pltpu.sync_copy(wv_v, w01_hbm.at[:, pl.ds(sid * tpt, tpt)])


def _make_route(mesh):
    return pl.kernel(
        _route_body,
        out_type=(
            jax.ShapeDtypeStruct((K, N), _f32),
            jax.ShapeDtypeStruct((K, N), _i32),
            jax.ShapeDtypeStruct((GBUF,), _i32),
        ),
        mesh=mesh,
        scratch_types=[
            pltpu.VMEM((K, N), _i32),
            pltpu.VMEM((K, N // NS), _i32),
            pltpu.VMEM((N // NS, NE), _f32),
            pltpu.VMEM((K, N // NS), _i32),
            pltpu.VMEM((K, N // NS), _f32),
            pltpu.VMEM((NS, L), _i32),
            pltpu.VMEM((L,), _i32),
            pltpu.VMEM((L,), _i32),
            pltpu.VMEM((L,), _i32),
            pltpu.VMEM((GBUF,), _i32),
        ],
        compiler_params=_SC_PARAMS,
    )


_B_CH = 16
_B_NCH = N // (NC * NS) // _B_CH


def _dispatch_body(pos_hbm, hs_hbm, hsort_hbm, i0_v, i1_v, rows_v, sem):
    wid = lax.axis_index("s") * NC + lax.axis_index("c")
    tb = wid * (N // (NC * NS))
    for j in range(_B_NCH):
        pltpu.sync_copy(pos_hbm.at[0, pl.ds(tb + j * _B_CH, _B_CH)], i0_v.at[j])
        pltpu.sync_copy(pos_hbm.at[1, pl.ds(tb + j * _B_CH, _B_CH)], i1_v.at[j])
    for j in range(_B_NCH):
        i0_v[j] = jnp.clip(i0_v[j], 0, ROWS - 1)
        i1_v[j] = jnp.clip(i1_v[j], 0, ROWS - 1)
    for j in range(_B_NCH):
        pltpu.sync_copy(hs_hbm.at[pl.ds(tb + j * _B_CH, _B_CH), :], rows_v)
        pltpu.async_copy(rows_v, hsort_hbm.at[i0_v.at[j]], sem).wait()
        pltpu.async_copy(rows_v, hsort_hbm.at[i1_v.at[j]], sem).wait()


def _make_dispatch(mesh):
    return pl.kernel(
        _dispatch_body,
        out_type=jax.ShapeDtypeStruct((ROWS, H), _f32),
        mesh=mesh,
        scratch_types=[
            pltpu.VMEM((_B_NCH, _B_CH), _i32),
            pltpu.VMEM((_B_NCH, _B_CH), _i32),
            pltpu.VMEM((_B_CH, H), _f32),
            pltpu.SemaphoreType.DMA,
        ],
        compiler_params=_SC_PARAMS,
    )


def _ffn_body(be_ref, hs_ref, wgu_ref, wd_ref, o_ref):
    g = pl.program_id(0)

    @pl.when(g < be_ref[48])
    def _():
        x = hs_ref[...].astype(jnp.bfloat16)
        gu = jnp.dot(x, wgu_ref[0], preferred_element_type=_f32)
        gate = gu[:, :F]
        up = gu[:, F:]
        act = (gate / (1.0 + jnp.exp(-gate))) * up
        o_ref[...] = jnp.dot(act.astype(jnp.bfloat16), wd_ref[0],
                             preferred_element_type=_f32)


_D_CH = 16
_D_NCH = N // (NC * NS) // _D_CH


def _combine_body(pos_hbm, osort_hbm, g0_hbm, g1_hbm, i0_v, i1_v, acc_v, sem):
    wid = lax.axis_index("s") * NC + lax.axis_index("c")
    tb = wid * (N // (NC * NS))
    for j in range(_D_NCH):
        pltpu.sync_copy(pos_hbm.at[0, pl.ds(tb + j * _D_CH, _D_CH)], i0_v.at[j])
        pltpu.sync_copy(pos_hbm.at[1, pl.ds(tb + j * _D_CH, _D_CH)], i1_v.at[j])
    for j in range(_D_NCH):
        i0_v[j] = jnp.clip(i0_v[j], 0, ROWS - 1)
        i1_v[j] = jnp.clip(i1_v[j], 0, ROWS - 1)
    for j in range(_D_NCH):
        pltpu.async_copy(osort_hbm.at[i0_v.at[j]], acc_v, sem).wait()
        pltpu.sync_copy(acc_v, g0_hbm.at[pl.ds(tb + j * _D_CH, _D_CH), :])
        pltpu.async_copy(osort_hbm.at[i1_v.at[j]], acc_v, sem).wait()
        pltpu.sync_copy(acc_v, g1_hbm.at[pl.ds(tb + j * _D_CH, _D_CH), :])


def _make_combine(mesh):
    return pl.kernel(
        _combine_body,
        out_type=(
            jax.ShapeDtypeStruct((N, H), _f32),
            jax.ShapeDtypeStruct((N, H), _f32),
        ),
        mesh=mesh,
        scratch_types=[
            pltpu.VMEM((_D_NCH, _D_CH), _i32),
            pltpu.VMEM((_D_NCH, _D_CH), _i32),
            pltpu.VMEM((_D_CH, H), _f32),
            pltpu.SemaphoreType.DMA,
        ],
        compiler_params=_SC_PARAMS,
    )


_E_BT = 256


def _mix_body(g0_ref, g1_ref, w_ref, o_ref):
    w0 = w_ref[0, 0, 0]
    w1 = w_ref[1, 0, 0]
    o_ref[...] = g0_ref[...] * w0[:, None] + g1_ref[...] * w1[:, None]


def kernel(hidden_states, routing_weights, router_indices, gate_up_proj, down_proj):
    b, s, h = hidden_states.shape
    hs2 = hidden_states.reshape(N, H)
    rw2 = routing_weights.reshape(N, NE)
    ri2 = router_indices.reshape(N, K).astype(_i32).T

    mesh = plsc.VectorSubcoreMesh(core_axis_name="c", subcore_axis_name="s")

    w01, pos, be = _make_route(mesh)(ri2, rw2)
    hsort = _make_dispatch(mesh)(pos, hs2)

    grid_spec = pltpu.PrefetchScalarGridSpec(
        num_scalar_prefetch=1,
        grid=(G,),
        in_specs=[
            pl.BlockSpec((BT, H), lambda g, be: (g, 0)),
            pl.BlockSpec((1, H, 2 * F), lambda g, be: (be[g], 0, 0)),
            pl.BlockSpec((1, F, H), lambda g, be: (be[g], 0, 0)),
        ],
        out_specs=pl.BlockSpec((BT, H), lambda g, be: (g, 0)),
    )
    osort = pl.pallas_call(
        _ffn_body,
        grid_spec=grid_spec,
        out_shape=jax.ShapeDtypeStruct((ROWS, H), _f32),
        compiler_params=pltpu.CompilerParams(vmem_limit_bytes=100 * 1024 * 1024),
    )(be, hsort, gate_up_proj.astype(jnp.bfloat16), down_proj.astype(jnp.bfloat16))

    g0, g1 = _make_combine(mesh)(pos, osort)

    final = pl.pallas_call(
        _mix_body,
        grid=(N // _E_BT,),
        in_specs=[
            pl.BlockSpec((_E_BT, H), lambda t: (t, 0)),
            pl.BlockSpec((_E_BT, H), lambda t: (t, 0)),
            pl.BlockSpec((K, 1, 1, _E_BT), lambda t: (0, t, 0, 0)),
        ],
        out_specs=pl.BlockSpec((_E_BT, H), lambda t: (t, 0)),
        out_shape=jax.ShapeDtypeStruct((N, H), _f32),
    )(g0, g1, w01.reshape(K, N // _E_BT, 1, _E_BT))
    return final.reshape(b, s, h)


# speedup vs baseline: 1.0870x; 1.0870x over previous
import jax
import jax.numpy as jnp
from jax import lax
from jax.experimental import pallas as pl
from jax.experimental.pallas import tpu as pltpu
from jax.experimental.pallas import tpu_sc as plsc

NE = 8
H = 2048
F = 768
K = 2
N = 4096
NPAIR = N * K
BT = 256
ROWS = NPAIR + NE * BT
G = ROWS // BT
GBUF = 64

NC = 2
NS = 16
L = 16
RPT = ROWS // NS

_i32 = jnp.int32
_f32 = jnp.float32
_SC_PARAMS = pltpu.CompilerParams(needs_layout_passes=False)


def _route_body(ri_hbm, rw_hbm, w01_hbm, pos_hbm, be_hbm,
                ri_full, ri_v, rw_v, pos_v, wv_v,
                cnt16_v, mybase_v, pb_v, cumblk_v, bev):
    cid = lax.axis_index("c")
    sid = lax.axis_index("s")
    iota = lax.iota(_i32, L)
    ones_i = jnp.ones((L,), _i32)
    zeros_i = jnp.zeros((L,), _i32)
    zeros_f = jnp.zeros((L,), _f32)

    @pl.when(cid == 0)
    def _():
        tpt = N // NS
        pltpu.sync_copy(ri_hbm, ri_full)
        pltpu.sync_copy(ri_hbm.at[:, pl.ds(sid * tpt, tpt)], ri_v)
        pltpu.sync_copy(rw_hbm.at[pl.ds(sid * tpt, tpt), :], rw_v)
        for s2 in range(NS):
            cnt16_v[s2] = zeros_i
        for s2 in range(NS):
            for k in range(K):
                for c in range(tpt // L):
                    ev = ri_full[k, pl.ds(s2 * tpt + c * L, L)]
                    plsc.addupdate_scatter(cnt16_v.at[s2], [ev], ones_i)

        total = zeros_i
        before = zeros_i
        for s2 in range(NS):
            row = cnt16_v[s2]
            total = total + row
            before = before + jnp.where(s2 < sid, row, zeros_i)
        padded = jnp.bitwise_and(total + (BT - 1), jnp.full((L,), -BT, _i32))
        cum = plsc.cumsum(padded)
        start = cum - padded
        mybase_v[...] = start + before
        pb_v[...] = lax.shift_right_logical(start, 8)
        cumblk_v[...] = lax.shift_right_logical(cum, 8)

        @pl.when(sid == 0)
        def _():
            nblk = plsc.load_gather(cumblk_v, [jnp.full((L,), 7, _i32)])
            for cb in range(GBUF // L):
                g_vec = iota + cb * L
                acc = zeros_i
                for e in range(1, NE):
                    pbe = plsc.load_gather(pb_v, [jnp.full((L,), e, _i32)])
                    acc = acc + (g_vec >= pbe).astype(_i32)
                acc = jnp.where(g_vec == 48, nblk, acc)
                bev[pl.ds(cb * L, L)] = acc
            pltpu.sync_copy(bev, be_hbm)

        for k in range(K):
            for c in range(tpt // L):
                ev = ri_v[k, pl.ds(c * L, L)]
                base = plsc.load_gather(mybase_v, [ev])
                r = zeros_i
                for e in range(NE):
                    mi = (ev == e).astype(_i32)
                    r = r + mi * (plsc.cumsum(mi) - 1)
                dest = jnp.clip(base + r, 0, ROWS - 1)
                plsc.addupdate_scatter(mybase_v, [ev], ones_i)
                pos_v[k, pl.ds(c * L, L)] = dest
                tokl = iota + c * L
                w = plsc.load_gather(rw_v, [tokl, ev])
                if k == 1:
                    ev0 = ri_v[0, pl.ds(c * L, L)]
                    w = jnp.where(ev != ev0, w, 0.0)
                wv_v[k, pl.ds(c * L, L)] = w
        pltpu.sync_copy(pos_v, pos_hbm.at[:, pl.ds(sid * tpt, tpt)])
        pltpu.sync_copy(wv_v, w01_hbm.at[:, pl.ds(sid * tpt, tpt)])


def _make_route(mesh):
    return pl.kernel(
        _route_body,
        out_type=(
            jax.ShapeDtypeStruct((K, N), _f32),
            jax.ShapeDtypeStruct((K, N), _i32),
            jax.ShapeDtypeStruct((GBUF,), _i32),
        ),
        mesh=mesh,
        scratch_types=[
            pltpu.VMEM((K, N), _i32),
            pltpu.VMEM((K, N // NS), _i32),
            pltpu.VMEM((N // NS, NE), _f32),
            pltpu.VMEM((K, N // NS), _i32),
            pltpu.VMEM((K, N // NS), _f32),
            pltpu.VMEM((NS, L), _i32),
            pltpu.VMEM((L,), _i32),
            pltpu.VMEM((L,), _i32),
            pltpu.VMEM((L,), _i32),
            pltpu.VMEM((GBUF,), _i32),
        ],
        compiler_params=_SC_PARAMS,
    )


_B_CH = 16
_B_NCH = N // (NC * NS) // _B_CH


def _dispatch_body(pos_hbm, hs_hbm, hsort_hbm, i0_v, i1_v, rows_v, sem):
    wid = lax.axis_index("s") * NC + lax.axis_index("c")
    tb = wid * (N // (NC * NS))
    for j in range(_B_NCH):
        pltpu.sync_copy(pos_hbm.at[0, pl.ds(tb + j * _B_CH, _B_CH)], i0_v.at[j])
        pltpu.sync_copy(pos_hbm.at[1, pl.ds(tb + j * _B_CH, _B_CH)], i1_v.at[j])
    for j in range(_B_NCH):
        i0_v[j] = jnp.clip(i0_v[j], 0, ROWS - 1)
        i1_v[j] = jnp.clip(i1_v[j], 0, ROWS - 1)
    for j in range(_B_NCH):
        pltpu.sync_copy(hs_hbm.at[pl.ds(tb + j * _B_CH, _B_CH), :], rows_v)
        pltpu.async_copy(rows_v, hsort_hbm.at[i0_v.at[j]], sem).wait()
        pltpu.async_copy(rows_v, hsort_hbm.at[i1_v.at[j]], sem).wait()


def _make_dispatch(mesh):
    return pl.kernel(
        _dispatch_body,
        out_type=jax.ShapeDtypeStruct((ROWS, H), _f32),
        mesh=mesh,
        scratch_types=[
            pltpu.VMEM((_B_NCH, _B_CH), _i32),
            pltpu.VMEM((_B_NCH, _B_CH), _i32),
            pltpu.VMEM((_B_CH, H), _f32),
            pltpu.SemaphoreType.DMA,
        ],
        compiler_params=_SC_PARAMS,
    )


def _ffn_body(be_ref, hs_ref, wgu_ref, wd_ref, o_ref):
    g = pl.program_id(0)

    @pl.when(g < be_ref[48])
    def _():
        x = hs_ref[...]
        gu = jnp.dot(x, wgu_ref[0], preferred_element_type=_f32)
        gate = gu[:, :F]
        up = gu[:, F:]
        act = (gate / (1.0 + jnp.exp(-gate))) * up
        o_ref[...] = jnp.dot(act, wd_ref[0], preferred_element_type=_f32)


_D_CH = 16
_D_NCH = N // (NC * NS) // _D_CH


def _combine_body(pos_hbm, osort_hbm, g0_hbm, g1_hbm, i0_v, i1_v, acc_v, sem):
    wid = lax.axis_index("s") * NC + lax.axis_index("c")
    tb = wid * (N // (NC * NS))
    for j in range(_D_NCH):
        pltpu.sync_copy(pos_hbm.at[0, pl.ds(tb + j * _D_CH, _D_CH)], i0_v.at[j])
        pltpu.sync_copy(pos_hbm.at[1, pl.ds(tb + j * _D_CH, _D_CH)], i1_v.at[j])
    for j in range(_D_NCH):
        i0_v[j] = jnp.clip(i0_v[j], 0, ROWS - 1)
        i1_v[j] = jnp.clip(i1_v[j], 0, ROWS - 1)
    for j in range(_D_NCH):
        pltpu.async_copy(osort_hbm.at[i0_v.at[j]], acc_v, sem).wait()
        pltpu.sync_copy(acc_v, g0_hbm.at[pl.ds(tb + j * _D_CH, _D_CH), :])
        pltpu.async_copy(osort_hbm.at[i1_v.at[j]], acc_v, sem).wait()
        pltpu.sync_copy(acc_v, g1_hbm.at[pl.ds(tb + j * _D_CH, _D_CH), :])


def _make_combine(mesh):
    return pl.kernel(
        _combine_body,
        out_type=(
            jax.ShapeDtypeStruct((N, H), _f32),
            jax.ShapeDtypeStruct((N, H), _f32),
        ),
        mesh=mesh,
        scratch_types=[
            pltpu.VMEM((_D_NCH, _D_CH), _i32),
            pltpu.VMEM((_D_NCH, _D_CH), _i32),
            pltpu.VMEM((_D_CH, H), _f32),
            pltpu.SemaphoreType.DMA,
        ],
        compiler_params=_SC_PARAMS,
    )


_E_BT = 256


def _mix_body(g0_ref, g1_ref, w_ref, o_ref):
    w0 = w_ref[0, 0, 0]
    w1 = w_ref[1, 0, 0]
    o_ref[...] = g0_ref[...] * w0[:, None] + g1_ref[...] * w1[:, None]


def kernel(hidden_states, routing_weights, router_indices, gate_up_proj, down_proj):
    b, s, h = hidden_states.shape
    hs2 = hidden_states.reshape(N, H)
    rw2 = routing_weights.reshape(N, NE)
    ri2 = router_indices.reshape(N, K).astype(_i32).T

    mesh = plsc.VectorSubcoreMesh(core_axis_name="c", subcore_axis_name="s")

    w01, pos, be = _make_route(mesh)(ri2, rw2)
    hsort = _make_dispatch(mesh)(pos, hs2)

    grid_spec = pltpu.PrefetchScalarGridSpec(
        num_scalar_prefetch=1,
        grid=(G,),
        in_specs=[
            pl.BlockSpec((BT, H), lambda g, be: (g, 0)),
            pl.BlockSpec((1, H, 2 * F), lambda g, be: (be[g], 0, 0)),
            pl.BlockSpec((1, F, H), lambda g, be: (be[g], 0, 0)),
        ],
        out_specs=pl.BlockSpec((BT, H), lambda g, be: (g, 0)),
    )
    osort = pl.pallas_call(
        _ffn_body,
        grid_spec=grid_spec,
        out_shape=jax.ShapeDtypeStruct((ROWS, H), _f32),
        compiler_params=pltpu.CompilerParams(vmem_limit_bytes=100 * 1024 * 1024),
    )(be, hsort, gate_up_proj, down_proj)

    g0, g1 = _make_combine(mesh)(pos, osort)

    final = pl.pallas_call(
        _mix_body,
        grid=(N // _E_BT,),
        in_specs=[
            pl.BlockSpec((_E_BT, H), lambda t: (t, 0)),
            pl.BlockSpec((_E_BT, H), lambda t: (t, 0)),
            pl.BlockSpec((K, 1, 1, _E_BT), lambda t: (0, t, 0, 0)),
        ],
        out_specs=pl.BlockSpec((_E_BT, H), lambda t: (t, 0)),
        out_shape=jax.ShapeDtypeStruct((N, H), _f32),
    )(g0, g1, w01.reshape(K, N // _E_BT, 1, _E_BT))
    return final.reshape(b, s, h)
